# baseline (speedup 1.0000x reference)
import functools

import jax
import jax.numpy as jnp
from jax import lax
from jax.experimental import pallas as pl
from jax.experimental.pallas import tpu as pltpu
from jax.experimental.pallas import tpu_sc as plsc

N = 10000
E = 320000
D = 128
H = 128
O = 128
EPS = 1e-5

BE = 2000

NC = 2
NS = 16
NW = NC * NS
EW = E // NW
CG = 400

_SC_MESH = plsc.VectorSubcoreMesh(
    core_axis_name="c", subcore_axis_name="s", num_cores=NC, num_subcores=NS)


def _gather_add_body(a_hbm, b_hbm, dst_hbm, src_hbm, p_hbm,
                     idx_d, idx_s, rows, sem):
    wid = lax.axis_index("s") * NC + lax.axis_index("c")
    base = wid * EW

    @pl.loop(0, EW // CG)
    def _chunk(i):
        off = base + i * CG
        pltpu.sync_copy(dst_hbm.at[pl.ds(off, CG)], idx_d)
        pltpu.sync_copy(src_hbm.at[pl.ds(off, CG)], idx_s)
        pltpu.async_copy(a_hbm.at[idx_d], rows, sem).wait()
        pltpu.async_copy(b_hbm.at[idx_s], rows, sem, add=True).wait()
        pltpu.sync_copy(rows, p_hbm.at[pl.ds(off, CG)])


FG = 8
NQ = 4
EQ = E // NQ
SCC = 800
NCHUNK = EQ // SCC
GPC = SCC // 16
PCW = 16000
NEP = 1000
MINF2 = -8323200


def _bf16max(a_i32, b_i32):
    a = plsc.bitcast(a_i32, jnp.bfloat16)
    b = plsc.bitcast(b_i32, jnp.bfloat16)
    return plsc.bitcast(jnp.maximum(a, b), jnp.int32)


def _scatter_max_body(h_hbm, dst_hbm, out_hbm, xch_hbm,
                      m_v, hb0, hb1, ib0, ib1, pbuf, olo, ohi,
                      semh0, semh1, semi0, semi1):
    c = lax.axis_index("c")
    s = lax.axis_index("s")
    g = s >> 2
    gg = c * 4 + g
    gcol = gg * FG
    q = s & 3
    wid = c * NS + s
    iota = lax.iota(jnp.int32, 16)
    rowpat = iota >> 3
    colpat = iota & 7
    ge8 = iota >= 8
    perm8 = (iota + 8) & 15
    iota8 = iota * 8
    perm8x8 = perm8 * 8
    colvs = [(iota + r) & 7 for r in range(FG)]
    minf = jnp.full((16,), MINF2, jnp.int32)
    hbufs = (hb0, hb1)
    ibufs = (ib0, ib1)
    semhs = (semh0, semh1)
    semis = (semi0, semi1)

    @pl.loop(0, N * FG // 16)
    def _init(j):
        m_v[pl.ds(j * 16, 16)] = minf

    def _issue(cidx, b):
        off = q * EQ + cidx * SCC
        pltpu.async_copy(dst_hbm.at[pl.ds(off, SCC)], ibufs[b], semis[b])
        pltpu.async_copy(h_hbm.at[pl.ds(off, SCC), pl.ds(gcol, FG)],
                         hbufs[b], semhs[b])

    _issue(0, 0)
    _issue(1, 1)

    @pl.loop(0, NCHUNK // 2)
    def _chunk(i):
        for b in range(2):
            cidx = i * 2 + b
            off = q * EQ + cidx * SCC
            ib = ibufs[b]
            hb = hbufs[b]
            pltpu.make_async_copy(dst_hbm.at[pl.ds(off, SCC)], ib,
                                  semis[b]).wait()
            pltpu.make_async_copy(h_hbm.at[pl.ds(off, SCC), pl.ds(gcol, FG)],
                                  hb, semhs[b]).wait()

            @pl.loop(0, GPC)
            def _group(j):
                dst16 = ib[pl.ds(j * 16, 16)]
                dst8 = dst16 * 8
                rowv = j * 16 + iota
                rowr = j * 16 + perm8
                dstr = plsc.load_gather(ib, [rowr])
                eq = dst16 == dstr
                smask = jnp.logical_not(jnp.logical_and(eq, ge8))
                hvs = [plsc.load_gather(hb, [rowv, colvs[r]])
                       for r in range(FG)]
                hrs = [plsc.load_gather(hb, [rowr, colvs[r]])
                       for r in range(FG)]
                for r in range(FG):
                    hv2 = jnp.where(eq, _bf16max(hvs[r], hrs[r]), hvs[r])
                    cur = plsc.load_gather(m_v, [dst8 + colvs[r]])
                    plsc.store_scatter(m_v, [dst8 + colvs[r]],
                                       _bf16max(cur, hv2), mask=smask)

            nxt = cidx + 2

            @pl.when(nxt < NCHUNK)
            def _prefetch():
                _issue(nxt, b)

    @pl.when(q != 0)
    def _publish():
        pltpu.sync_copy(m_v, xch_hbm.at[wid])

    plsc.subcore_barrier()

    @pl.when(q == 0)
    def _merge():
        for t in (1, 2, 3):
            @pl.loop(0, N * FG // PCW)
            def _mch(k):
                pltpu.sync_copy(xch_hbm.at[wid + t, pl.ds(k * PCW, PCW)],
                                pbuf)

                @pl.loop(0, PCW // 16)
                def _mvec(j):
                    idx = k * PCW + j * 16
                    m_v[pl.ds(idx, 16)] = _bf16max(m_v[pl.ds(idx, 16)],
                                                   pbuf[pl.ds(j * 16, 16)])

        @pl.loop(0, N // NEP)
        def _ep(k):
            @pl.loop(0, NEP * FG // 16)
            def _ev(j):
                w = m_v[pl.ds(k * NEP * FG + j * 16, 16)]
                lo = jnp.maximum(
                    plsc.bitcast(jnp.left_shift(w, 16), jnp.float32), 0.0)
                hi = jnp.maximum(
                    plsc.bitcast(w & jnp.int32(-65536), jnp.float32), 0.0)
                plsc.store_scatter(olo, [j * 2 + rowpat, colpat], lo)
                plsc.store_scatter(ohi, [j * 2 + rowpat, colpat], hi)

            pltpu.sync_copy(olo, out_hbm.at[pl.ds(k * NEP, NEP),
                                            pl.ds(gcol, FG)])
            pltpu.sync_copy(ohi, out_hbm.at[pl.ds(k * NEP, NEP),
                                            pl.ds(64 + gcol, FG)])


_scatter_max = functools.partial(
    pl.kernel,
    out_type=(
        jax.ShapeDtypeStruct((N, O), jnp.float32),
        jax.ShapeDtypeStruct((NW, N * FG), jnp.int32),
    ),
    mesh=_SC_MESH,
    scratch_types=[
        pltpu.VMEM((N * FG,), jnp.int32),
        pltpu.VMEM((SCC, FG), jnp.int32),
        pltpu.VMEM((SCC, FG), jnp.int32),
        pltpu.VMEM((SCC,), jnp.int32),
        pltpu.VMEM((SCC,), jnp.int32),
        pltpu.VMEM((PCW,), jnp.int32),
        pltpu.VMEM((NEP, FG), jnp.float32),
        pltpu.VMEM((NEP, FG), jnp.float32),
        pltpu.SemaphoreType.DMA,
        pltpu.SemaphoreType.DMA,
        pltpu.SemaphoreType.DMA,
        pltpu.SemaphoreType.DMA,
    ],
    compiler_params=pltpu.CompilerParams(
        use_tc_tiling_on_sc=False, needs_layout_passes=False),
)(_scatter_max_body)


_gather_add = functools.partial(
    pl.kernel,
    out_type=jax.ShapeDtypeStruct((E, H), jnp.float32),
    mesh=_SC_MESH,
    scratch_types=[
        pltpu.VMEM((CG,), jnp.int32),
        pltpu.VMEM((CG,), jnp.int32),
        pltpu.VMEM((CG, H), jnp.float32),
        pltpu.SemaphoreType.DMA,
    ],
)(_gather_add_body)


def _node_tables_kernel(x_ref, gamma_ref, beta_ref, w1d_ref, w1b_ref, b1_ref,
                        a_ref, b_ref):
    x = x_ref[...]
    mean = jnp.mean(x, axis=0, keepdims=True)
    var = jnp.mean((x - mean) ** 2, axis=0, keepdims=True)
    scale = gamma_ref[...] * jax.lax.rsqrt(var + EPS)
    xn = (x - mean) * scale + beta_ref[...]
    a_ref[...] = jnp.dot(xn, w1d_ref[...], preferred_element_type=jnp.float32) + b1_ref[...]
    b_ref[...] = jnp.dot(xn, w1b_ref[...], preferred_element_type=jnp.float32)


def _edge_mlp_kernel(p_ref, w2_ref, b2_ref, h_ref):
    p = jnp.maximum(p_ref[...], 0.0)
    h = jnp.dot(p, w2_ref[...], preferred_element_type=jnp.float32) + b2_ref[...]
    u = lax.bitcast_convert_type(h, jnp.uint32) + 0x8000
    col = lax.broadcasted_iota(jnp.int32, (BE, O), 1)
    wfull = jnp.where(col < 64, u >> 16, u & jnp.uint32(0xFFFF0000))
    wor = wfull | pltpu.roll(wfull, 64, 1)
    h_ref[...] = lax.bitcast_convert_type(wor[:, :64], jnp.int32)


def kernel(x, edge_index, gamma, beta, W1, b1, W2, b2):
    w1d = W1[:D] - W1[D:]
    w1b = W1[D:]
    a_tab, b_tab = pl.pallas_call(
        _node_tables_kernel,
        out_shape=(
            jax.ShapeDtypeStruct((N, H), jnp.float32),
            jax.ShapeDtypeStruct((N, H), jnp.float32),
        ),
    )(x, gamma.reshape(1, D), beta.reshape(1, D), w1d, w1b, b1.reshape(1, H))

    src = edge_index[0]
    dst = edge_index[1]
    p = _gather_add(a_tab, b_tab, dst, src)

    h = pl.pallas_call(
        _edge_mlp_kernel,
        grid=(E // BE,),
        in_specs=[
            pl.BlockSpec((BE, H), lambda i: (i, 0)),
            pl.BlockSpec((H, O), lambda i: (0, 0)),
            pl.BlockSpec((1, O), lambda i: (0, 0)),
        ],
        out_specs=pl.BlockSpec((BE, O // 2), lambda i: (i, 0)),
        out_shape=jax.ShapeDtypeStruct((E, O // 2), jnp.int32),
    )(p, W2, b2.reshape(1, O))

    out, _ = _scatter_max(h, dst)
    return out


# speedup vs baseline: 1.0645x; 1.0645x over previous
import functools

import jax
import jax.numpy as jnp
from jax import lax
from jax.experimental import pallas as pl
from jax.experimental.pallas import tpu as pltpu
from jax.experimental.pallas import tpu_sc as plsc

N = 10000
E = 320000
D = 128
H = 128
O = 128
EPS = 1e-5

BE = 2000

NC = 2
NS = 16
NW = NC * NS
EW = E // NW
CG = 400

_SC_MESH = plsc.VectorSubcoreMesh(
    core_axis_name="c", subcore_axis_name="s", num_cores=NC, num_subcores=NS)


NCHG = EW // CG


def _gather_add_body(a_hbm, b_hbm, dst_hbm, src_hbm, p_hbm,
                     id0, id1, is0, is1, rw0, rw1,
                     semid0, semid1, semis0, semis1,
                     sema0, sema1, semb0, semb1, semw0, semw1):
    wid = lax.axis_index("s") * NC + lax.axis_index("c")
    base = wid * EW
    idd = (id0, id1)
    iss = (is0, is1)
    rws = (rw0, rw1)
    semid = (semid0, semid1)
    semis = (semis0, semis1)
    sema = (sema0, sema1)
    semb = (semb0, semb1)
    semw = (semw0, semw1)

    def _sl(c):
        return pl.ds(base + c * CG, CG)

    def _issue_idx(c, b):
        pltpu.async_copy(dst_hbm.at[_sl(c)], idd[b], semid[b])
        pltpu.async_copy(src_hbm.at[_sl(c)], iss[b], semis[b])

    _issue_idx(0, 0)
    _issue_idx(1, 1)
    pltpu.make_async_copy(dst_hbm.at[_sl(0)], idd[0], semid[0]).wait()
    pltpu.make_async_copy(src_hbm.at[_sl(0)], iss[0], semis[0]).wait()
    pltpu.async_copy(a_hbm.at[idd[0]], rws[0], sema[0])

    def _step(c, b):
        ob = 1 - b
        pltpu.make_async_copy(a_hbm.at[idd[b]], rws[b], sema[b]).wait()
        pltpu.async_copy(b_hbm.at[iss[b]], rws[b], semb[b], add=True)

        @pl.when(c >= 1)
        def _wout_prev():
            pltpu.make_async_copy(rws[ob], p_hbm.at[_sl(c - 1)],
                                  semw[ob]).wait()

        @pl.when(c + 1 < NCHG)
        def _next_a():
            pltpu.make_async_copy(dst_hbm.at[_sl(c + 1)], idd[ob],
                                  semid[ob]).wait()
            pltpu.make_async_copy(src_hbm.at[_sl(c + 1)], iss[ob],
                                  semis[ob]).wait()
            pltpu.async_copy(a_hbm.at[idd[ob]], rws[ob], sema[ob])

        pltpu.make_async_copy(b_hbm.at[iss[b]], rws[b], semb[b]).wait()
        pltpu.async_copy(rws[b], p_hbm.at[_sl(c)], semw[b])

        @pl.when(c + 2 < NCHG)
        def _next_idx():
            _issue_idx(c + 2, b)

    @pl.loop(0, NCHG // 2)
    def _chunk(j):
        _step(j * 2, 0)
        _step(j * 2 + 1, 1)

    c = NCHG - 1
    pltpu.make_async_copy(a_hbm.at[idd[0]], rws[0], sema[0]).wait()
    pltpu.async_copy(b_hbm.at[iss[0]], rws[0], semb[0], add=True).wait()
    pltpu.make_async_copy(rws[1], p_hbm.at[_sl(c - 1)], semw[1]).wait()
    pltpu.sync_copy(rws[0], p_hbm.at[_sl(c)])


FG = 8
NQ = 4
EQ = E // NQ
SCC = 800
NCHUNK = EQ // SCC
GPC = SCC // 16
PCW = 16000
NEP = 1000
MINF2 = -8323200


def _bf16max(a_i32, b_i32):
    a = plsc.bitcast(a_i32, jnp.bfloat16)
    b = plsc.bitcast(b_i32, jnp.bfloat16)
    return plsc.bitcast(jnp.maximum(a, b), jnp.int32)


def _scatter_max_body(h_hbm, dst_hbm, out_hbm, xch_hbm,
                      m_v, hb0, hb1, ib0, ib1, pbuf, olo, ohi,
                      semh0, semh1, semi0, semi1):
    c = lax.axis_index("c")
    s = lax.axis_index("s")
    g = s >> 2
    gg = c * 4 + g
    gcol = gg * FG
    q = s & 3
    wid = c * NS + s
    iota = lax.iota(jnp.int32, 16)
    rowpat = iota >> 3
    colpat = iota & 7
    ge8 = iota >= 8
    perm8 = (iota + 8) & 15
    iota8 = iota * 8
    perm8x8 = perm8 * 8
    colvs = [(iota + r) & 7 for r in range(FG)]
    minf = jnp.full((16,), MINF2, jnp.int32)
    hbufs = (hb0, hb1)
    ibufs = (ib0, ib1)
    semhs = (semh0, semh1)
    semis = (semi0, semi1)

    @pl.loop(0, N * FG // 16)
    def _init(j):
        m_v[pl.ds(j * 16, 16)] = minf

    def _issue(cidx, b):
        off = q * EQ + cidx * SCC
        pltpu.async_copy(dst_hbm.at[pl.ds(off, SCC)], ibufs[b], semis[b])
        pltpu.async_copy(h_hbm.at[pl.ds(off, SCC), pl.ds(gcol, FG)],
                         hbufs[b], semhs[b])

    _issue(0, 0)
    _issue(1, 1)

    @pl.loop(0, NCHUNK // 2)
    def _chunk(i):
        for b in range(2):
            cidx = i * 2 + b
            off = q * EQ + cidx * SCC
            ib = ibufs[b]
            hb = hbufs[b]
            pltpu.make_async_copy(dst_hbm.at[pl.ds(off, SCC)], ib,
                                  semis[b]).wait()
            pltpu.make_async_copy(h_hbm.at[pl.ds(off, SCC), pl.ds(gcol, FG)],
                                  hb, semhs[b]).wait()

            @pl.loop(0, GPC)
            def _group(j):
                dst16 = ib[pl.ds(j * 16, 16)]
                dst8 = dst16 * 8
                rowv = j * 16 + iota
                rowr = j * 16 + perm8
                dstr = plsc.load_gather(ib, [rowr])
                eq = dst16 == dstr
                smask = jnp.logical_not(jnp.logical_and(eq, ge8))
                hvs = [plsc.load_gather(hb, [rowv, colvs[r]])
                       for r in range(FG)]
                hrs = [plsc.load_gather(hb, [rowr, colvs[r]])
                       for r in range(FG)]
                for r in range(FG):
                    hv2 = jnp.where(eq, _bf16max(hvs[r], hrs[r]), hvs[r])
                    cur = plsc.load_gather(m_v, [dst8 + colvs[r]])
                    plsc.store_scatter(m_v, [dst8 + colvs[r]],
                                       _bf16max(cur, hv2), mask=smask)

            nxt = cidx + 2

            @pl.when(nxt < NCHUNK)
            def _prefetch():
                _issue(nxt, b)

    @pl.when(q != 0)
    def _publish():
        pltpu.sync_copy(m_v, xch_hbm.at[wid])

    plsc.subcore_barrier()

    @pl.when(q == 0)
    def _merge():
        for t in (1, 2, 3):
            @pl.loop(0, N * FG // PCW)
            def _mch(k):
                pltpu.sync_copy(xch_hbm.at[wid + t, pl.ds(k * PCW, PCW)],
                                pbuf)

                @pl.loop(0, PCW // 16)
                def _mvec(j):
                    idx = k * PCW + j * 16
                    m_v[pl.ds(idx, 16)] = _bf16max(m_v[pl.ds(idx, 16)],
                                                   pbuf[pl.ds(j * 16, 16)])

        @pl.loop(0, N // NEP)
        def _ep(k):
            @pl.loop(0, NEP * FG // 16)
            def _ev(j):
                w = m_v[pl.ds(k * NEP * FG + j * 16, 16)]
                lo = jnp.maximum(
                    plsc.bitcast(jnp.left_shift(w, 16), jnp.float32), 0.0)
                hi = jnp.maximum(
                    plsc.bitcast(w & jnp.int32(-65536), jnp.float32), 0.0)
                plsc.store_scatter(olo, [j * 2 + rowpat, colpat], lo)
                plsc.store_scatter(ohi, [j * 2 + rowpat, colpat], hi)

            pltpu.sync_copy(olo, out_hbm.at[pl.ds(k * NEP, NEP),
                                            pl.ds(gcol, FG)])
            pltpu.sync_copy(ohi, out_hbm.at[pl.ds(k * NEP, NEP),
                                            pl.ds(64 + gcol, FG)])


_scatter_max = functools.partial(
    pl.kernel,
    out_type=(
        jax.ShapeDtypeStruct((N, O), jnp.float32),
        jax.ShapeDtypeStruct((NW, N * FG), jnp.int32),
    ),
    mesh=_SC_MESH,
    scratch_types=[
        pltpu.VMEM((N * FG,), jnp.int32),
        pltpu.VMEM((SCC, FG), jnp.int32),
        pltpu.VMEM((SCC, FG), jnp.int32),
        pltpu.VMEM((SCC,), jnp.int32),
        pltpu.VMEM((SCC,), jnp.int32),
        pltpu.VMEM((PCW,), jnp.int32),
        pltpu.VMEM((NEP, FG), jnp.float32),
        pltpu.VMEM((NEP, FG), jnp.float32),
        pltpu.SemaphoreType.DMA,
        pltpu.SemaphoreType.DMA,
        pltpu.SemaphoreType.DMA,
        pltpu.SemaphoreType.DMA,
    ],
    compiler_params=pltpu.CompilerParams(
        use_tc_tiling_on_sc=False, needs_layout_passes=False),
)(_scatter_max_body)


_gather_add = functools.partial(
    pl.kernel,
    out_type=jax.ShapeDtypeStruct((E, H), jnp.float32),
    mesh=_SC_MESH,
    scratch_types=[
        pltpu.VMEM((CG,), jnp.int32),
        pltpu.VMEM((CG,), jnp.int32),
        pltpu.VMEM((CG,), jnp.int32),
        pltpu.VMEM((CG,), jnp.int32),
        pltpu.VMEM((CG, H), jnp.float32),
        pltpu.VMEM((CG, H), jnp.float32),
    ] + [pltpu.SemaphoreType.DMA] * 10,
)(_gather_add_body)


def _node_tables_kernel(x_ref, gamma_ref, beta_ref, w1d_ref, w1b_ref, b1_ref,
                        a_ref, b_ref):
    x = x_ref[...]
    mean = jnp.mean(x, axis=0, keepdims=True)
    var = jnp.mean((x - mean) ** 2, axis=0, keepdims=True)
    scale = gamma_ref[...] * jax.lax.rsqrt(var + EPS)
    xn = (x - mean) * scale + beta_ref[...]
    a_ref[...] = jnp.dot(xn, w1d_ref[...], preferred_element_type=jnp.float32) + b1_ref[...]
    b_ref[...] = jnp.dot(xn, w1b_ref[...], preferred_element_type=jnp.float32)


def _edge_mlp_kernel(p_ref, w2_ref, b2_ref, h_ref):
    p = jnp.maximum(p_ref[...], 0.0)
    h = jnp.dot(p, w2_ref[...], preferred_element_type=jnp.float32) + b2_ref[...]
    rl = lax.bitcast_convert_type(h[:, :64], jnp.uint32) + 0x8000
    rh = lax.bitcast_convert_type(h[:, 64:], jnp.uint32) + 0x8000
    h_ref[...] = lax.bitcast_convert_type(
        (rl >> 16) | (rh & jnp.uint32(0xFFFF0000)), jnp.int32)


def kernel(x, edge_index, gamma, beta, W1, b1, W2, b2):
    w1d = W1[:D] - W1[D:]
    w1b = W1[D:]
    a_tab, b_tab = pl.pallas_call(
        _node_tables_kernel,
        out_shape=(
            jax.ShapeDtypeStruct((N, H), jnp.float32),
            jax.ShapeDtypeStruct((N, H), jnp.float32),
        ),
    )(x, gamma.reshape(1, D), beta.reshape(1, D), w1d, w1b, b1.reshape(1, H))

    src = edge_index[0]
    dst = edge_index[1]
    p = _gather_add(a_tab, b_tab, dst, src)

    h = pl.pallas_call(
        _edge_mlp_kernel,
        grid=(E // BE,),
        in_specs=[
            pl.BlockSpec((BE, H), lambda i: (i, 0)),
            pl.BlockSpec((H, O), lambda i: (0, 0)),
            pl.BlockSpec((1, O), lambda i: (0, 0)),
        ],
        out_specs=pl.BlockSpec((BE, O // 2), lambda i: (i, 0)),
        out_shape=jax.ShapeDtypeStruct((E, O // 2), jnp.int32),
    )(p, W2, b2.reshape(1, O))

    out, _ = _scatter_max(h, dst)
    return out


# speedup vs baseline: 1.1910x; 1.1188x over previous
import functools

import jax
import jax.numpy as jnp
from jax import lax
from jax.experimental import pallas as pl
from jax.experimental.pallas import tpu as pltpu
from jax.experimental.pallas import tpu_sc as plsc

N = 10000
E = 320000
D = 128
H = 128
O = 128
EPS = 1e-5

BE = 2000

NC = 2
NS = 16
NW = NC * NS
EW = E // NW
CG = 400

_SC_MESH = plsc.VectorSubcoreMesh(
    core_axis_name="c", subcore_axis_name="s", num_cores=NC, num_subcores=NS)


NCHG = EW // CG


def _gather_add_body(a_hbm, b_hbm, dst_hbm, src_hbm, p_hbm,
                     id0, id1, is0, is1, rw0, rw1,
                     semid0, semid1, semis0, semis1,
                     sema0, sema1, semb0, semb1, semw0, semw1):
    wid = lax.axis_index("s") * NC + lax.axis_index("c")
    base = wid * EW
    idd = (id0, id1)
    iss = (is0, is1)
    rws = (rw0, rw1)
    semid = (semid0, semid1)
    semis = (semis0, semis1)
    sema = (sema0, sema1)
    semb = (semb0, semb1)
    semw = (semw0, semw1)

    def _sl(c):
        return pl.ds(base + c * CG, CG)

    def _issue_idx(c, b):
        pltpu.async_copy(dst_hbm.at[_sl(c)], idd[b], semid[b])
        pltpu.async_copy(src_hbm.at[_sl(c)], iss[b], semis[b])

    _issue_idx(0, 0)
    _issue_idx(1, 1)
    pltpu.make_async_copy(dst_hbm.at[_sl(0)], idd[0], semid[0]).wait()
    pltpu.make_async_copy(src_hbm.at[_sl(0)], iss[0], semis[0]).wait()
    pltpu.async_copy(a_hbm.at[idd[0]], rws[0], sema[0])

    def _step(c, b):
        ob = 1 - b
        pltpu.make_async_copy(a_hbm.at[idd[b]], rws[b], sema[b]).wait()
        pltpu.async_copy(b_hbm.at[iss[b]], rws[b], semb[b], add=True)

        @pl.when(c >= 1)
        def _wout_prev():
            pltpu.make_async_copy(rws[ob], p_hbm.at[_sl(c - 1)],
                                  semw[ob]).wait()

        @pl.when(c + 1 < NCHG)
        def _next_a():
            pltpu.make_async_copy(dst_hbm.at[_sl(c + 1)], idd[ob],
                                  semid[ob]).wait()
            pltpu.make_async_copy(src_hbm.at[_sl(c + 1)], iss[ob],
                                  semis[ob]).wait()
            pltpu.async_copy(a_hbm.at[idd[ob]], rws[ob], sema[ob])

        pltpu.make_async_copy(b_hbm.at[iss[b]], rws[b], semb[b]).wait()
        pltpu.async_copy(rws[b], p_hbm.at[_sl(c)], semw[b])

        @pl.when(c + 2 < NCHG)
        def _next_idx():
            _issue_idx(c + 2, b)

    @pl.loop(0, NCHG // 2)
    def _chunk(j):
        _step(j * 2, 0)
        _step(j * 2 + 1, 1)

    c = NCHG - 1
    pltpu.make_async_copy(a_hbm.at[idd[0]], rws[0], sema[0]).wait()
    pltpu.async_copy(b_hbm.at[iss[0]], rws[0], semb[0], add=True).wait()
    pltpu.make_async_copy(rws[1], p_hbm.at[_sl(c - 1)], semw[1]).wait()
    pltpu.sync_copy(rws[0], p_hbm.at[_sl(c)])


FG = 8
NQ = 4
EQ = E // NQ
SCC = 800
NCHUNK = EQ // SCC
GPC = SCC // 16
PCW = 20000
NEP = 500
MINF2 = -8323200


def _bf16max(a_i32, b_i32):
    a = plsc.bitcast(a_i32, jnp.bfloat16)
    b = plsc.bitcast(b_i32, jnp.bfloat16)
    return plsc.bitcast(jnp.maximum(a, b), jnp.int32)


def _scatter_max_body(h_hbm, dst_hbm, out_hbm, xch_hbm,
                      m_v, hb0, hb1, ib0, ib1, pbuf, olo, ohi,
                      semh0, semh1, semi0, semi1):
    c = lax.axis_index("c")
    s = lax.axis_index("s")
    g = s >> 2
    gg = c * 4 + g
    gcol = gg * FG
    q = s & 3
    wid = c * NS + s
    iota = lax.iota(jnp.int32, 16)
    rowpat = iota >> 3
    colpat = iota & 7
    ge8 = iota >= 8
    perm8 = (iota + 8) & 15
    iota8 = iota * 8
    perm8x8 = perm8 * 8
    colvs = [(iota + r) & 7 for r in range(FG)]
    minf = jnp.full((16,), MINF2, jnp.int32)
    hbufs = (hb0, hb1)
    ibufs = (ib0, ib1)
    semhs = (semh0, semh1)
    semis = (semi0, semi1)

    @pl.loop(0, N * FG // 16)
    def _init(j):
        m_v[pl.ds(j * 16, 16)] = minf

    def _issue(cidx, b):
        off = q * EQ + cidx * SCC
        pltpu.async_copy(dst_hbm.at[pl.ds(off, SCC)], ibufs[b], semis[b])
        pltpu.async_copy(h_hbm.at[pl.ds(off, SCC), pl.ds(gcol, FG)],
                         hbufs[b], semhs[b])

    _issue(0, 0)
    _issue(1, 1)

    @pl.loop(0, NCHUNK // 2)
    def _chunk(i):
        for b in range(2):
            cidx = i * 2 + b
            off = q * EQ + cidx * SCC
            ib = ibufs[b]
            hb = hbufs[b]
            pltpu.make_async_copy(dst_hbm.at[pl.ds(off, SCC)], ib,
                                  semis[b]).wait()
            pltpu.make_async_copy(h_hbm.at[pl.ds(off, SCC), pl.ds(gcol, FG)],
                                  hb, semhs[b]).wait()

            @pl.loop(0, GPC)
            def _group(j):
                dst16 = ib[pl.ds(j * 16, 16)]
                dst8 = dst16 * 8
                rowv = j * 16 + iota
                rowr = j * 16 + perm8
                dstr = plsc.load_gather(ib, [rowr])
                eq = dst16 == dstr
                smask = jnp.logical_not(jnp.logical_and(eq, ge8))
                hvs = [plsc.load_gather(hb, [rowv, colvs[r]])
                       for r in range(FG)]
                hrs = [plsc.load_gather(hb, [rowr, colvs[r]])
                       for r in range(FG)]
                for r in range(FG):
                    hv2 = jnp.where(eq, _bf16max(hvs[r], hrs[r]), hvs[r])
                    cur = plsc.load_gather(m_v, [dst8 + colvs[r]])
                    plsc.store_scatter(m_v, [dst8 + colvs[r]],
                                       _bf16max(cur, hv2), mask=smask)

            nxt = cidx + 2

            @pl.when(nxt < NCHUNK)
            def _prefetch():
                _issue(nxt, b)

    pltpu.sync_copy(m_v, xch_hbm.at[wid])
    plsc.subcore_barrier()

    row0 = q * (N // NQ)
    word0 = row0 * FG
    for t in (1, 2, 3):
        pw = wid - q + ((q + t) & 3)
        pltpu.sync_copy(xch_hbm.at[pw, pl.ds(word0, PCW)], pbuf)

        @pl.loop(0, PCW // 16)
        def _mvec(j):
            idx = word0 + j * 16
            m_v[pl.ds(idx, 16)] = _bf16max(m_v[pl.ds(idx, 16)],
                                           pbuf[pl.ds(j * 16, 16)])

    @pl.loop(0, N // NQ // NEP)
    def _ep(k):
        @pl.loop(0, NEP * FG // 16)
        def _ev(j):
            w = m_v[pl.ds(word0 + k * NEP * FG + j * 16, 16)]
            lo = jnp.maximum(
                plsc.bitcast(jnp.left_shift(w, 16), jnp.float32), 0.0)
            hi = jnp.maximum(
                plsc.bitcast(w & jnp.int32(-65536), jnp.float32), 0.0)
            plsc.store_scatter(olo, [j * 2 + rowpat, colpat], lo)
            plsc.store_scatter(ohi, [j * 2 + rowpat, colpat], hi)

        pltpu.sync_copy(olo, out_hbm.at[pl.ds(row0 + k * NEP, NEP),
                                        pl.ds(gcol, FG)])
        pltpu.sync_copy(ohi, out_hbm.at[pl.ds(row0 + k * NEP, NEP),
                                        pl.ds(64 + gcol, FG)])


_scatter_max = functools.partial(
    pl.kernel,
    out_type=(
        jax.ShapeDtypeStruct((N, O), jnp.float32),
        jax.ShapeDtypeStruct((NW, N * FG), jnp.int32),
    ),
    mesh=_SC_MESH,
    scratch_types=[
        pltpu.VMEM((N * FG,), jnp.int32),
        pltpu.VMEM((SCC, FG), jnp.int32),
        pltpu.VMEM((SCC, FG), jnp.int32),
        pltpu.VMEM((SCC,), jnp.int32),
        pltpu.VMEM((SCC,), jnp.int32),
        pltpu.VMEM((PCW,), jnp.int32),
        pltpu.VMEM((NEP, FG), jnp.float32),
        pltpu.VMEM((NEP, FG), jnp.float32),
        pltpu.SemaphoreType.DMA,
        pltpu.SemaphoreType.DMA,
        pltpu.SemaphoreType.DMA,
        pltpu.SemaphoreType.DMA,
    ],
    compiler_params=pltpu.CompilerParams(
        use_tc_tiling_on_sc=False, needs_layout_passes=False),
)(_scatter_max_body)


_gather_add = functools.partial(
    pl.kernel,
    out_type=jax.ShapeDtypeStruct((E, H), jnp.float32),
    mesh=_SC_MESH,
    scratch_types=[
        pltpu.VMEM((CG,), jnp.int32),
        pltpu.VMEM((CG,), jnp.int32),
        pltpu.VMEM((CG,), jnp.int32),
        pltpu.VMEM((CG,), jnp.int32),
        pltpu.VMEM((CG, H), jnp.float32),
        pltpu.VMEM((CG, H), jnp.float32),
    ] + [pltpu.SemaphoreType.DMA] * 10,
)(_gather_add_body)


def _node_tables_kernel(x_ref, gamma_ref, beta_ref, w1d_ref, w1b_ref, b1_ref,
                        a_ref, b_ref):
    x = x_ref[...]
    mean = jnp.mean(x, axis=0, keepdims=True)
    var = jnp.mean((x - mean) ** 2, axis=0, keepdims=True)
    scale = gamma_ref[...] * jax.lax.rsqrt(var + EPS)
    xn = (x - mean) * scale + beta_ref[...]
    a_ref[...] = jnp.dot(xn, w1d_ref[...], preferred_element_type=jnp.float32) + b1_ref[...]
    b_ref[...] = jnp.dot(xn, w1b_ref[...], preferred_element_type=jnp.float32)


def _edge_mlp_kernel(p_ref, w2_ref, b2_ref, h_ref):
    p = jnp.maximum(p_ref[...], 0.0)
    h = jnp.dot(p, w2_ref[...], preferred_element_type=jnp.float32) + b2_ref[...]
    rl = lax.bitcast_convert_type(h[:, :64], jnp.uint32) + 0x8000
    rh = lax.bitcast_convert_type(h[:, 64:], jnp.uint32) + 0x8000
    h_ref[...] = lax.bitcast_convert_type(
        (rl >> 16) | (rh & jnp.uint32(0xFFFF0000)), jnp.int32)


def kernel(x, edge_index, gamma, beta, W1, b1, W2, b2):
    w1d = W1[:D] - W1[D:]
    w1b = W1[D:]
    a_tab, b_tab = pl.pallas_call(
        _node_tables_kernel,
        out_shape=(
            jax.ShapeDtypeStruct((N, H), jnp.float32),
            jax.ShapeDtypeStruct((N, H), jnp.float32),
        ),
    )(x, gamma.reshape(1, D), beta.reshape(1, D), w1d, w1b, b1.reshape(1, H))

    src = edge_index[0]
    dst = edge_index[1]
    p = _gather_add(a_tab, b_tab, dst, src)

    h = pl.pallas_call(
        _edge_mlp_kernel,
        grid=(E // BE,),
        in_specs=[
            pl.BlockSpec((BE, H), lambda i: (i, 0)),
            pl.BlockSpec((H, O), lambda i: (0, 0)),
            pl.BlockSpec((1, O), lambda i: (0, 0)),
        ],
        out_specs=pl.BlockSpec((BE, O // 2), lambda i: (i, 0)),
        out_shape=jax.ShapeDtypeStruct((E, O // 2), jnp.int32),
    )(p, W2, b2.reshape(1, O))

    out, _ = _scatter_max(h, dst)
    return out


# speedup vs baseline: 1.1974x; 1.0054x over previous
import functools

import jax
import jax.numpy as jnp
from jax import lax
from jax.experimental import pallas as pl
from jax.experimental.pallas import tpu as pltpu
from jax.experimental.pallas import tpu_sc as plsc

N = 10000
E = 320000
D = 128
H = 128
O = 128
EPS = 1e-5

BE = 2000

NC = 2
NS = 16
NW = NC * NS
EW = E // NW
CG = 400

_SC_MESH = plsc.VectorSubcoreMesh(
    core_axis_name="c", subcore_axis_name="s", num_cores=NC, num_subcores=NS)


NCHG = EW // CG


def _gather_add_body(a_hbm, b_hbm, dst_hbm, src_hbm, p_hbm,
                     id0, id1, is0, is1, rw0, rw1,
                     semid0, semid1, semis0, semis1,
                     sema0, sema1, semb0, semb1, semw0, semw1):
    wid = lax.axis_index("s") * NC + lax.axis_index("c")
    base = wid * EW
    idd = (id0, id1)
    iss = (is0, is1)
    rws = (rw0, rw1)
    semid = (semid0, semid1)
    semis = (semis0, semis1)
    sema = (sema0, sema1)
    semb = (semb0, semb1)
    semw = (semw0, semw1)

    def _sl(c):
        return pl.ds(base + c * CG, CG)

    def _issue_idx(c, b):
        pltpu.async_copy(dst_hbm.at[_sl(c)], idd[b], semid[b])
        pltpu.async_copy(src_hbm.at[_sl(c)], iss[b], semis[b])

    _issue_idx(0, 0)
    _issue_idx(1, 1)
    pltpu.make_async_copy(dst_hbm.at[_sl(0)], idd[0], semid[0]).wait()
    pltpu.make_async_copy(src_hbm.at[_sl(0)], iss[0], semis[0]).wait()
    pltpu.async_copy(a_hbm.at[idd[0]], rws[0], sema[0])

    def _step(c, b):
        ob = 1 - b
        pltpu.make_async_copy(a_hbm.at[idd[b]], rws[b], sema[b]).wait()
        pltpu.async_copy(b_hbm.at[iss[b]], rws[b], semb[b], add=True)

        @pl.when(c >= 1)
        def _wout_prev():
            pltpu.make_async_copy(rws[ob], p_hbm.at[_sl(c - 1)],
                                  semw[ob]).wait()

        @pl.when(c + 1 < NCHG)
        def _next_a():
            pltpu.make_async_copy(dst_hbm.at[_sl(c + 1)], idd[ob],
                                  semid[ob]).wait()
            pltpu.make_async_copy(src_hbm.at[_sl(c + 1)], iss[ob],
                                  semis[ob]).wait()
            pltpu.async_copy(a_hbm.at[idd[ob]], rws[ob], sema[ob])

        pltpu.make_async_copy(b_hbm.at[iss[b]], rws[b], semb[b]).wait()
        pltpu.async_copy(rws[b], p_hbm.at[_sl(c)], semw[b])

        @pl.when(c + 2 < NCHG)
        def _next_idx():
            _issue_idx(c + 2, b)

    @pl.loop(0, NCHG // 2)
    def _chunk(j):
        _step(j * 2, 0)
        _step(j * 2 + 1, 1)

    c = NCHG - 1
    pltpu.make_async_copy(a_hbm.at[idd[0]], rws[0], sema[0]).wait()
    pltpu.async_copy(b_hbm.at[iss[0]], rws[0], semb[0], add=True).wait()
    pltpu.make_async_copy(rws[1], p_hbm.at[_sl(c - 1)], semw[1]).wait()
    pltpu.sync_copy(rws[0], p_hbm.at[_sl(c)])


FG = 8
NQ = 4
EQ = E // NQ
SCC = 800
NCHUNK = EQ // SCC
GPC = SCC // 16
PCW = 20000
NEP = 500
MINF2 = -8323200


def _bf16max(a_i32, b_i32):
    a = plsc.bitcast(a_i32, jnp.bfloat16)
    b = plsc.bitcast(b_i32, jnp.bfloat16)
    return plsc.bitcast(jnp.maximum(a, b), jnp.int32)


def _scatter_max_body(h_hbm, dst_hbm, out_hbm, xch_hbm,
                      m_v, hb0, hb1, ib0, ib1, pbuf, olo, ohi,
                      semh0, semh1, semi0, semi1):
    c = lax.axis_index("c")
    s = lax.axis_index("s")
    g = s >> 2
    gg = c * 4 + g
    gcol = gg * FG
    q = s & 3
    wid = c * NS + s
    iota = lax.iota(jnp.int32, 16)
    rowpat = iota >> 3
    colpat = iota & 7
    ge8 = iota >= 8
    perm8 = (iota + 8) & 15
    iota8 = iota * 8
    perm8x8 = perm8 * 8
    colvs = [(iota + r) & 7 for r in range(FG)]
    minf = jnp.full((16,), MINF2, jnp.int32)
    hbufs = (hb0, hb1)
    ibufs = (ib0, ib1)
    semhs = (semh0, semh1)
    semis = (semi0, semi1)

    @pl.loop(0, N * FG // 16)
    def _init(j):
        m_v[pl.ds(j * 16, 16)] = minf

    def _issue(cidx, b):
        off = q * EQ + cidx * SCC
        pltpu.async_copy(dst_hbm.at[pl.ds(off, SCC)], ibufs[b], semis[b])
        pltpu.async_copy(h_hbm.at[pl.ds(off, SCC), pl.ds(gcol, FG)],
                         hbufs[b], semhs[b])

    _issue(0, 0)
    _issue(1, 1)

    @pl.loop(0, NCHUNK // 2)
    def _chunk(i):
        for b in range(2):
            cidx = i * 2 + b
            off = q * EQ + cidx * SCC
            ib = ibufs[b]
            hb = hbufs[b]
            pltpu.make_async_copy(dst_hbm.at[pl.ds(off, SCC)], ib,
                                  semis[b]).wait()
            pltpu.make_async_copy(h_hbm.at[pl.ds(off, SCC), pl.ds(gcol, FG)],
                                  hb, semhs[b]).wait()

            @pl.loop(0, GPC, unroll=2)
            def _group(j):
                dst16 = ib[pl.ds(j * 16, 16)]
                dst8 = dst16 * 8
                rowv = j * 16 + iota
                rowr = j * 16 + perm8
                dstr = plsc.load_gather(ib, [rowr])
                eq = dst16 == dstr
                smask = jnp.logical_not(jnp.logical_and(eq, ge8))
                hvs = [plsc.load_gather(hb, [rowv, colvs[r]])
                       for r in range(FG)]
                hrs = [plsc.load_gather(hb, [rowr, colvs[r]])
                       for r in range(FG)]
                for r in range(FG):
                    hv2 = jnp.where(eq, _bf16max(hvs[r], hrs[r]), hvs[r])
                    cur = plsc.load_gather(m_v, [dst8 + colvs[r]])
                    plsc.store_scatter(m_v, [dst8 + colvs[r]],
                                       _bf16max(cur, hv2), mask=smask)

            nxt = cidx + 2

            @pl.when(nxt < NCHUNK)
            def _prefetch():
                _issue(nxt, b)

    pltpu.sync_copy(m_v, xch_hbm.at[wid])
    plsc.subcore_barrier()

    row0 = q * (N // NQ)
    word0 = row0 * FG
    for t in (1, 2, 3):
        pw = wid - q + ((q + t) & 3)
        pltpu.sync_copy(xch_hbm.at[pw, pl.ds(word0, PCW)], pbuf)

        @pl.loop(0, PCW // 16)
        def _mvec(j):
            idx = word0 + j * 16
            m_v[pl.ds(idx, 16)] = _bf16max(m_v[pl.ds(idx, 16)],
                                           pbuf[pl.ds(j * 16, 16)])

    @pl.loop(0, N // NQ // NEP)
    def _ep(k):
        @pl.loop(0, NEP * FG // 16)
        def _ev(j):
            w = m_v[pl.ds(word0 + k * NEP * FG + j * 16, 16)]
            lo = jnp.maximum(
                plsc.bitcast(jnp.left_shift(w, 16), jnp.float32), 0.0)
            hi = jnp.maximum(
                plsc.bitcast(w & jnp.int32(-65536), jnp.float32), 0.0)
            plsc.store_scatter(olo, [j * 2 + rowpat, colpat], lo)
            plsc.store_scatter(ohi, [j * 2 + rowpat, colpat], hi)

        pltpu.sync_copy(olo, out_hbm.at[pl.ds(row0 + k * NEP, NEP),
                                        pl.ds(gcol, FG)])
        pltpu.sync_copy(ohi, out_hbm.at[pl.ds(row0 + k * NEP, NEP),
                                        pl.ds(64 + gcol, FG)])


_scatter_max = functools.partial(
    pl.kernel,
    out_type=(
        jax.ShapeDtypeStruct((N, O), jnp.float32),
        jax.ShapeDtypeStruct((NW, N * FG), jnp.int32),
    ),
    mesh=_SC_MESH,
    scratch_types=[
        pltpu.VMEM((N * FG,), jnp.int32),
        pltpu.VMEM((SCC, FG), jnp.int32),
        pltpu.VMEM((SCC, FG), jnp.int32),
        pltpu.VMEM((SCC,), jnp.int32),
        pltpu.VMEM((SCC,), jnp.int32),
        pltpu.VMEM((PCW,), jnp.int32),
        pltpu.VMEM((NEP, FG), jnp.float32),
        pltpu.VMEM((NEP, FG), jnp.float32),
        pltpu.SemaphoreType.DMA,
        pltpu.SemaphoreType.DMA,
        pltpu.SemaphoreType.DMA,
        pltpu.SemaphoreType.DMA,
    ],
    compiler_params=pltpu.CompilerParams(
        use_tc_tiling_on_sc=False, needs_layout_passes=False),
)(_scatter_max_body)


_gather_add = functools.partial(
    pl.kernel,
    out_type=jax.ShapeDtypeStruct((E, H), jnp.float32),
    mesh=_SC_MESH,
    scratch_types=[
        pltpu.VMEM((CG,), jnp.int32),
        pltpu.VMEM((CG,), jnp.int32),
        pltpu.VMEM((CG,), jnp.int32),
        pltpu.VMEM((CG,), jnp.int32),
        pltpu.VMEM((CG, H), jnp.float32),
        pltpu.VMEM((CG, H), jnp.float32),
    ] + [pltpu.SemaphoreType.DMA] * 10,
)(_gather_add_body)


def _node_tables_kernel(x_ref, gamma_ref, beta_ref, w1d_ref, w1b_ref, b1_ref,
                        a_ref, b_ref):
    x = x_ref[...]
    mean = jnp.mean(x, axis=0, keepdims=True)
    var = jnp.mean((x - mean) ** 2, axis=0, keepdims=True)
    scale = gamma_ref[...] * jax.lax.rsqrt(var + EPS)
    xn = (x - mean) * scale + beta_ref[...]
    a_ref[...] = jnp.dot(xn, w1d_ref[...], preferred_element_type=jnp.float32) + b1_ref[...]
    b_ref[...] = jnp.dot(xn, w1b_ref[...], preferred_element_type=jnp.float32)


def _edge_mlp_kernel(p_ref, w2_ref, b2_ref, h_ref):
    p = jnp.maximum(p_ref[...], 0.0)
    h = jnp.dot(p, w2_ref[...], preferred_element_type=jnp.float32) + b2_ref[...]
    rl = lax.bitcast_convert_type(h[:, :64], jnp.uint32) + 0x8000
    rh = lax.bitcast_convert_type(h[:, 64:], jnp.uint32) + 0x8000
    h_ref[...] = lax.bitcast_convert_type(
        (rl >> 16) | (rh & jnp.uint32(0xFFFF0000)), jnp.int32)


def kernel(x, edge_index, gamma, beta, W1, b1, W2, b2):
    w1d = W1[:D] - W1[D:]
    w1b = W1[D:]
    a_tab, b_tab = pl.pallas_call(
        _node_tables_kernel,
        out_shape=(
            jax.ShapeDtypeStruct((N, H), jnp.float32),
            jax.ShapeDtypeStruct((N, H), jnp.float32),
        ),
    )(x, gamma.reshape(1, D), beta.reshape(1, D), w1d, w1b, b1.reshape(1, H))

    src = edge_index[0]
    dst = edge_index[1]
    p = _gather_add(a_tab, b_tab, dst, src)

    h = pl.pallas_call(
        _edge_mlp_kernel,
        grid=(E // BE,),
        in_specs=[
            pl.BlockSpec((BE, H), lambda i: (i, 0)),
            pl.BlockSpec((H, O), lambda i: (0, 0)),
            pl.BlockSpec((1, O), lambda i: (0, 0)),
        ],
        out_specs=pl.BlockSpec((BE, O // 2), lambda i: (i, 0)),
        out_shape=jax.ShapeDtypeStruct((E, O // 2), jnp.int32),
    )(p, W2, b2.reshape(1, O))

    out, _ = _scatter_max(h, dst)
    return out
